# Initial kernel scaffold; baseline (speedup 1.0000x reference)
import jax
import jax.numpy as jnp
from jax.experimental import pallas as pl


def kernel(pattern, force_x, force_y, z_vel, gaussian, noise_levels, flip_x, flip_y):
    raise NotImplementedError("write your pallas kernel here")


# speedup vs baseline: 4.6559x; 4.6559x over previous
import functools

import jax
import jax.numpy as jnp
from jax import lax
from jax.experimental import pallas as pl
from jax.experimental.pallas import tpu as pltpu
from jax.experimental.pallas import tpu_sc as plsc

B, H, W = 4096, 128, 128
NC, NS = 2, 16
NW = NC * NS
SPW = B // NW
L = 16
CPR = W // L

_mesh = plsc.VectorSubcoreMesh(core_axis_name="c", subcore_axis_name="s")

f32 = jnp.float32
i32 = jnp.int32


@functools.partial(
    pl.kernel,
    out_type=(
        jax.ShapeDtypeStruct((B, H, W), f32),
        jax.ShapeDtypeStruct((B,), f32),
        jax.ShapeDtypeStruct((B,), f32),
        jax.ShapeDtypeStruct((B, H, W), f32),
    ),
    mesh=_mesh,
    compiler_params=pltpu.CompilerParams(needs_layout_passes=False),
    scratch_types=[
        pltpu.VMEM((H, W), f32),
        pltpu.VMEM((H, W), f32),
        pltpu.VMEM((H, W), f32),
        pltpu.VMEM((SPW,), f32),
        pltpu.VMEM((SPW,), i32),
        pltpu.VMEM((SPW,), i32),
        pltpu.VMEM((SPW,), f32),
        pltpu.VMEM((SPW,), f32),
        pltpu.VMEM((SPW,), f32),
        pltpu.VMEM((SPW,), f32),
    ],
)
def _aug_sc(patt_hbm, fx_hbm, fy_hbm, zvel_hbm, gauss_hbm, nl_hbm,
            hm_hbm, wm_hbm, sgx_hbm, sgy_hbm,
            patt_out, fx_out, fy_out, zvel_out,
            patt_v, gauss_v, zvel_v, nl_v, hm_v, wm_v,
            sgx_v, sgy_v, fx_v, fy_v):
    wid = lax.axis_index("s") * NC + lax.axis_index("c")
    base = wid * SPW

    pltpu.sync_copy(nl_hbm.at[pl.ds(base, SPW)], nl_v)
    pltpu.sync_copy(hm_hbm.at[pl.ds(base, SPW)], hm_v)
    pltpu.sync_copy(wm_hbm.at[pl.ds(base, SPW)], wm_v)
    pltpu.sync_copy(sgx_hbm.at[pl.ds(base, SPW)], sgx_v)
    pltpu.sync_copy(sgy_hbm.at[pl.ds(base, SPW)], sgy_v)
    pltpu.sync_copy(fx_hbm.at[pl.ds(base, SPW)], fx_v)
    pltpu.sync_copy(fy_hbm.at[pl.ds(base, SPW)], fy_v)

    for c in range(SPW // L):
        sl = pl.ds(c * L, L)
        fx_v[sl] = fx_v[sl] * sgx_v[sl]
        fy_v[sl] = fy_v[sl] * sgy_v[sl]
    pltpu.sync_copy(fx_v, fx_out.at[pl.ds(base, SPW)])
    pltpu.sync_copy(fy_v, fy_out.at[pl.ds(base, SPW)])

    lane_iota = lax.iota(i32, L)

    def sample_body(s, _):
        b = base + s
        pltpu.sync_copy(patt_hbm.at[b], patt_v)
        pltpu.sync_copy(gauss_hbm.at[b], gauss_v)
        pltpu.sync_copy(zvel_hbm.at[b], zvel_v)

        idx_s = jnp.full((L,), s, dtype=i32)
        nl = plsc.load_gather(nl_v, [idx_s])
        one_m = 1.0 - nl
        hm = plsc.load_gather(hm_v, [idx_s])
        wm = plsc.load_gather(wm_v, [idx_s])
        cols = [lax.bitwise_xor(lane_iota + k * L, wm) for k in range(CPR)]

        def patt_row(i, _):
            row = lax.bitwise_xor(jnp.full((L,), i, dtype=i32), hm)
            for k in range(CPR):
                sl = pl.ds(k * L, L)
                p = plsc.load_gather(patt_v, [row, cols[k]])
                g = gauss_v[i, sl]
                gauss_v[i, sl] = one_m * p + nl * g
            return _

        def zvel_row(i, _):
            row = lax.bitwise_xor(jnp.full((L,), i, dtype=i32), hm)
            for k in range(CPR):
                z = plsc.load_gather(zvel_v, [row, cols[k]])
                patt_v[i, pl.ds(k * L, L)] = z
            return _

        lax.fori_loop(0, H, patt_row, 0)
        lax.fori_loop(0, H, zvel_row, 0)

        pltpu.sync_copy(gauss_v, patt_out.at[b])
        pltpu.sync_copy(patt_v, zvel_out.at[b])
        return _

    lax.fori_loop(0, SPW, sample_body, 0)


def kernel(pattern, force_x, force_y, z_vel, gaussian, noise_levels,
           flip_x, flip_y):
    hm = flip_x.astype(i32) * (H - 1)
    wm = flip_y.astype(i32) * (W - 1)
    sgx = jnp.where(flip_x, -1.0, 1.0).astype(f32)
    sgy = jnp.where(flip_y, -1.0, 1.0).astype(f32)
    patt, fx, fy, zv = _aug_sc(pattern, force_x, force_y, z_vel, gaussian,
                               noise_levels, hm, wm, sgx, sgy)
    return patt, fx, fy, zv


# speedup vs baseline: 10.2249x; 2.1961x over previous
import functools

import jax
import jax.numpy as jnp
from jax import lax
from jax.experimental import pallas as pl
from jax.experimental.pallas import tpu as pltpu
from jax.experimental.pallas import tpu_sc as plsc

B, H, W = 4096, 128, 128
NC, NS = 2, 16
NW = NC * NS
SPW = B // NW
L = 16
CPR = W // L
CH = 64
TURNS = SPW * (H // CH)
NSLOT = 2

_mesh = plsc.VectorSubcoreMesh(core_axis_name="c", subcore_axis_name="s")

f32 = jnp.float32
i32 = jnp.int32


@functools.partial(
    pl.kernel,
    out_type=(
        jax.ShapeDtypeStruct((B * H, W), f32),
        jax.ShapeDtypeStruct((B,), f32),
        jax.ShapeDtypeStruct((B,), f32),
        jax.ShapeDtypeStruct((B * H, W), f32),
    ),
    mesh=_mesh,
    compiler_params=pltpu.CompilerParams(needs_layout_passes=False),
    scratch_types=(
        [pltpu.VMEM((CH, W), f32) for _ in range(NSLOT)]
        + [pltpu.VMEM((CH, W), f32) for _ in range(NSLOT)]
        + [pltpu.VMEM((CH, W), f32) for _ in range(NSLOT)]
        + [pltpu.VMEM((CH, W), f32) for _ in range(NSLOT)]
        + [pltpu.VMEM((CH, W), f32) for _ in range(NSLOT)]
        + [
            pltpu.VMEM((SPW,), f32),
            pltpu.VMEM((SPW,), i32),
            pltpu.VMEM((SPW,), i32),
            pltpu.VMEM((SPW,), f32),
            pltpu.VMEM((SPW,), f32),
            pltpu.VMEM((SPW,), f32),
            pltpu.VMEM((SPW,), f32),
        ]
        + [pltpu.SemaphoreType.DMA for _ in range(5 * NSLOT)]
    ),
)
def _aug_sc(patt_hbm, fx_hbm, fy_hbm, zvel_hbm, gauss_hbm, nl_hbm,
            hm_hbm, wm_hbm, sgx_hbm, sgy_hbm,
            patt_out, fx_out, fy_out, zvel_out,
            pin0, pin1, gin0, gin1, zin0, zin1, pout0, pout1,
            zout0, zout1,
            nl_v, hm_v, wm_v, sgx_v, sgy_v, fx_v, fy_v,
            spi0, spi1, sgi0, sgi1, szi0, szi1, spo0, spo1, szo0, szo1):
    pin = (pin0, pin1)
    gin = (gin0, gin1)
    zin = (zin0, zin1)
    pout = (pout0, pout1)
    zout = (zout0, zout1)
    sem_pi = (spi0, spi1)
    sem_gi = (sgi0, sgi1)
    sem_zi = (szi0, szi1)
    sem_po = (spo0, spo1)
    sem_zo = (szo0, szo1)

    wid = lax.axis_index("s") * NC + lax.axis_index("c")
    base = wid * SPW

    pltpu.sync_copy(nl_hbm.at[pl.ds(base, SPW)], nl_v)
    pltpu.sync_copy(hm_hbm.at[pl.ds(base, SPW)], hm_v)
    pltpu.sync_copy(wm_hbm.at[pl.ds(base, SPW)], wm_v)
    pltpu.sync_copy(sgx_hbm.at[pl.ds(base, SPW)], sgx_v)
    pltpu.sync_copy(sgy_hbm.at[pl.ds(base, SPW)], sgy_v)
    pltpu.sync_copy(fx_hbm.at[pl.ds(base, SPW)], fx_v)
    pltpu.sync_copy(fy_hbm.at[pl.ds(base, SPW)], fy_v)

    for c in range(SPW // L):
        sl = pl.ds(c * L, L)
        fx_v[sl] = fx_v[sl] * sgx_v[sl]
        fy_v[sl] = fy_v[sl] * sgy_v[sl]
    pltpu.sync_copy(fx_v, fx_out.at[pl.ds(base, SPW)])
    pltpu.sync_copy(fy_v, fy_out.at[pl.ds(base, SPW)])

    lane_iota = lax.iota(i32, L)

    def splat(ref, s):
        return plsc.load_gather(ref, [jnp.full((L,), s, dtype=i32)])

    def hm_scalar(s):
        chunk = hm_v[pl.ds(pl.multiple_of((s // L) * L, L), L)]
        masked = jnp.where(lane_iota == s % L, chunk, 0)
        return jnp.sum(masked, axis=0)

    def issue_in(j, t):
        s = t // (H // CH)
        r0 = (t % (H // CH)) * CH
        g0 = pl.multiple_of((base + s) * H + r0, CH)
        hs = hm_scalar(s)
        p0 = (base + s) * H + lax.bitwise_xor(r0, lax.bitwise_and(hs, H - CH))
        p0 = pl.multiple_of(p0, CH)
        pltpu.make_async_copy(
            patt_hbm.at[pl.ds(p0, CH)], pin[j], sem_pi[j]).start()
        pltpu.make_async_copy(
            zvel_hbm.at[pl.ds(p0, CH)], zin[j], sem_zi[j]).start()
        pltpu.make_async_copy(
            gauss_hbm.at[pl.ds(g0, CH)], gin[j], sem_gi[j]).start()

    def wait_in(j):
        pltpu.make_async_copy(patt_hbm.at[pl.ds(0, CH)], pin[j],
                              sem_pi[j]).wait()
        pltpu.make_async_copy(zvel_hbm.at[pl.ds(0, CH)], zin[j],
                              sem_zi[j]).wait()
        pltpu.make_async_copy(gauss_hbm.at[pl.ds(0, CH)], gin[j],
                              sem_gi[j]).wait()

    def issue_out(j, t):
        s = t // (H // CH)
        r0 = (t % (H // CH)) * CH
        g0 = pl.multiple_of((base + s) * H + r0, CH)
        pltpu.make_async_copy(pout[j], patt_out.at[pl.ds(g0, CH)],
                              sem_po[j]).start()
        pltpu.make_async_copy(zout[j], zvel_out.at[pl.ds(g0, CH)],
                              sem_zo[j]).start()

    def wait_out(j):
        pltpu.make_async_copy(pout[j], patt_out.at[pl.ds(0, CH)],
                              sem_po[j]).wait()
        pltpu.make_async_copy(zout[j], zvel_out.at[pl.ds(0, CH)],
                              sem_zo[j]).wait()

    def compute(j, t):
        s = t // (H // CH)
        nl = splat(nl_v, s)
        one_m = 1.0 - nl
        fy_b = splat(wm_v, s) != 0
        rxor = lax.bitwise_and(hm_scalar(s), CH - 1)

        def row_body(r, _):
            rs = lax.bitwise_xor(r, rxor)
            p = [pin[j][rs, pl.ds(k * L, L)] for k in range(CPR)]
            z = [zin[j][rs, pl.ds(k * L, L)] for k in range(CPR)]
            for k in range(CPR):
                sl = pl.ds(k * L, L)
                ps = jnp.where(fy_b, jnp.flip(p[CPR - 1 - k], 0), p[k])
                zs = jnp.where(fy_b, jnp.flip(z[CPR - 1 - k], 0), z[k])
                pout[j][r, sl] = one_m * ps + nl * gin[j][r, sl]
                zout[j][r, sl] = zs
            return _

        lax.fori_loop(0, CH, row_body, 0)

    for j in range(NSLOT):
        issue_in(j, jnp.int32(j))

    def pair_body(t0, carry):
        for j in range(NSLOT):
            t = t0 + j
            wait_in(j)

            @pl.when(t >= NSLOT)
            def _():
                wait_out(j)

            compute(j, t)
            issue_out(j, t)

            @pl.when(t + NSLOT < TURNS)
            def _():
                issue_in(j, t + NSLOT)

        return carry

    lax.fori_loop(0, TURNS // NSLOT, lambda i, c: pair_body(i * NSLOT, c), 0)

    for j in range(NSLOT):
        wait_out(j)


def kernel(pattern, force_x, force_y, z_vel, gaussian, noise_levels,
           flip_x, flip_y):
    hm = flip_x.astype(i32) * (H - 1)
    wm = flip_y.astype(i32) * (W - 1)
    sgx = jnp.where(flip_x, -1.0, 1.0).astype(f32)
    sgy = jnp.where(flip_y, -1.0, 1.0).astype(f32)
    patt, fx, fy, zv = _aug_sc(
        pattern.reshape(B * H, W), force_x, force_y,
        z_vel.reshape(B * H, W), gaussian.reshape(B * H, W),
        noise_levels, hm, wm, sgx, sgy)
    return patt.reshape(B, H, W), fx, fy, zv.reshape(B, H, W)
